# Initial kernel scaffold; baseline (speedup 1.0000x reference)
import jax
import jax.numpy as jnp
from jax.experimental import pallas as pl


def kernel(tokens, emb_table):
    raise NotImplementedError("write your pallas kernel here")


# speedup vs baseline: 6.4783x; 6.4783x over previous
import functools

import jax
import jax.numpy as jnp
from jax import lax
from jax.experimental import pallas as pl
from jax.experimental.pallas import tpu as pltpu
from jax.experimental.pallas import tpu_sc as plsc

B = 1024
L = 200
D = 128
V = 73

NC = 2
NS = 16
NW = NC * NS
TPW = (B * L) // NW
CHUNK = 256
NCHUNK = TPW // CHUNK
IDX_ROWS = TPW // 128


def _positional_encoding():
    even_i = jnp.arange(0, D, 2, dtype=jnp.float32)
    denominator = jnp.power(10000.0, even_i / D)
    position = jnp.arange(L, dtype=jnp.float32).reshape(L, 1)
    even_pe = jnp.sin(position / denominator)
    odd_pe = jnp.cos(position / denominator)
    stacked = jnp.stack([even_pe, odd_pe], axis=2)
    return stacked.reshape(L, -1)


def _fuse_kernel(emb_ref, pos_ref, f_ref):
    f_ref[...] = emb_ref[...][None, :, :] + pos_ref[...][:, None, :]


def _build_fused_table(emb, pos):
    return pl.pallas_call(
        _fuse_kernel,
        out_shape=jax.ShapeDtypeStruct((L, V, D), jnp.float32),
    )(emb, pos)


def _sc_body(f_hbm, tok_hbm, off_hbm, out_hbm,
             tok_v, off_v, idx_v, buf0, buf1, sem_g, sem_w):
    wid = lax.axis_index("s") * NC + lax.axis_index("c")
    base = wid * TPW

    pltpu.sync_copy(tok_hbm.at[pl.ds(base, TPW)], tok_v)
    pltpu.sync_copy(off_hbm, off_v)

    def idx_row(r, _):
        for k in range(8):
            s = r * 128 + k * 16
            idx_v[r, pl.ds(k * 16, 16)] = (
                tok_v[pl.ds(s, 16)] + off_v[pl.ds(s, 16)]
            )
        return 0

    lax.fori_loop(0, IDX_ROWS, idx_row, 0)

    bufs = (buf0, buf1)
    writes = [None] * NCHUNK
    for c in range(NCHUNK):
        buf = bufs[c % 2]
        if c >= 2:
            writes[c - 2].wait()
        g0 = pltpu.async_copy(f_hbm.at[idx_v.at[2 * c]],
                              buf.at[pl.ds(0, 128)], sem_g)
        g1 = pltpu.async_copy(f_hbm.at[idx_v.at[2 * c + 1]],
                              buf.at[pl.ds(128, 128)], sem_g)
        g0.wait()
        g1.wait()
        writes[c] = pltpu.async_copy(
            buf, out_hbm.at[pl.ds(base + c * CHUNK, CHUNK)], sem_w)
    writes[NCHUNK - 2].wait()
    writes[NCHUNK - 1].wait()


@functools.partial(jax.jit, donate_argnums=())
def _run(tokens_flat, f_flat, off):
    mesh = plsc.VectorSubcoreMesh(core_axis_name="c", subcore_axis_name="s")
    gather = functools.partial(
        pl.kernel,
        mesh=mesh,
        out_type=jax.ShapeDtypeStruct((B * L, D), jnp.float32),
        scratch_types=[
            pltpu.VMEM((TPW,), jnp.int32),
            pltpu.VMEM((TPW,), jnp.int32),
            pltpu.VMEM((IDX_ROWS, 128), jnp.int32),
            pltpu.VMEM((CHUNK, D), jnp.float32),
            pltpu.VMEM((CHUNK, D), jnp.float32),
            pltpu.SemaphoreType.DMA,
            pltpu.SemaphoreType.DMA,
        ],
    )(_sc_body)
    return gather(f_flat, tokens_flat, off)


def kernel(tokens, emb_table):
    pos = _positional_encoding()
    f = _build_fused_table(emb_table, pos).reshape(L * V, D)
    tokens_flat = tokens.reshape(-1).astype(jnp.int32)
    off = (jnp.arange(TPW, dtype=jnp.int32) % L) * V
    out = _run(tokens_flat, f, off)
    return out.reshape(B, L, D)


# speedup vs baseline: 6.6278x; 1.0231x over previous
import functools

import jax
import jax.numpy as jnp
from jax import lax
from jax.experimental import pallas as pl
from jax.experimental.pallas import tpu as pltpu
from jax.experimental.pallas import tpu_sc as plsc

B = 1024
L = 200
D = 128
V = 73

NC = 2
NS = 16
NW = NC * NS
TPW = (B * L) // NW
CHUNK = 128
NCHUNK = TPW // CHUNK
NBUF = 6
IDX_ROWS = TPW // 128


def _positional_encoding():
    even_i = jnp.arange(0, D, 2, dtype=jnp.float32)
    denominator = jnp.power(10000.0, even_i / D)
    position = jnp.arange(L, dtype=jnp.float32).reshape(L, 1)
    even_pe = jnp.sin(position / denominator)
    odd_pe = jnp.cos(position / denominator)
    stacked = jnp.stack([even_pe, odd_pe], axis=2)
    return stacked.reshape(L, -1)


def _fuse_kernel(emb_ref, pos_ref, f_ref):
    f_ref[...] = emb_ref[...][None, :, :] + pos_ref[...][:, None, :]


def _build_fused_table(emb, pos):
    return pl.pallas_call(
        _fuse_kernel,
        out_shape=jax.ShapeDtypeStruct((L, V, D), jnp.float32),
    )(emb, pos)


def _sc_body(f_hbm, tok_hbm, off_hbm, out_hbm,
             tok_v, off_v, idx_v, bufs, sem_g, sem_w):
    wid = lax.axis_index("s") * NC + lax.axis_index("c")
    base = wid * TPW

    pltpu.sync_copy(tok_hbm.at[pl.ds(base, TPW)], tok_v)
    pltpu.sync_copy(off_hbm, off_v)

    def idx_row(r, _):
        for k in range(8):
            s = r * 128 + k * 16
            idx_v[r, pl.ds(k * 16, 16)] = (
                tok_v[pl.ds(s, 16)] + off_v[pl.ds(s, 16)]
            )
        return 0

    lax.fori_loop(0, IDX_ROWS, idx_row, 0)

    def gather(c):
        return pltpu.async_copy(f_hbm.at[idx_v.at[c]],
                                bufs.at[c % NBUF], sem_g.at[c % NBUF])

    gs = [None] * NCHUNK
    writes = [None] * NCHUNK
    for c in range(NBUF - 1):
        gs[c] = gather(c)
    for c in range(NCHUNK):
        gs[c].wait()
        writes[c] = pltpu.async_copy(
            bufs.at[c % NBUF], out_hbm.at[pl.ds(base + c * CHUNK, CHUNK)],
            sem_w.at[c % NBUF])
        if c + NBUF - 1 < NCHUNK:
            if c >= 1:
                writes[c - 1].wait()
            gs[c + NBUF - 1] = gather(c + NBUF - 1)
    for c in range(NCHUNK - NBUF, NCHUNK):
        writes[c].wait()


@functools.partial(jax.jit, donate_argnums=())
def _run(tokens_flat, f_flat, off):
    mesh = plsc.VectorSubcoreMesh(core_axis_name="c", subcore_axis_name="s")
    gather = functools.partial(
        pl.kernel,
        mesh=mesh,
        out_type=jax.ShapeDtypeStruct((B * L, D), jnp.float32),
        scratch_types=[
            pltpu.VMEM((TPW,), jnp.int32),
            pltpu.VMEM((TPW,), jnp.int32),
            pltpu.VMEM((IDX_ROWS, 128), jnp.int32),
            pltpu.VMEM((NBUF, CHUNK, D), jnp.float32),
            pltpu.SemaphoreType.DMA((NBUF,)),
            pltpu.SemaphoreType.DMA((NBUF,)),
        ],
    )(_sc_body)
    return gather(f_flat, tokens_flat, off)


def kernel(tokens, emb_table):
    pos = _positional_encoding()
    f = _build_fused_table(emb_table, pos).reshape(L * V, D)
    tokens_flat = tokens.reshape(-1).astype(jnp.int32)
    off = (jnp.arange(TPW, dtype=jnp.int32) % L) * V
    out = _run(tokens_flat, f, off)
    return out.reshape(B, L, D)
